# baseline (speedup 1.0000x reference)
import jax
import jax.numpy as jnp
from jax import lax
from jax.experimental import pallas as pl
from jax.experimental.pallas import tpu as pltpu
from jax.experimental.pallas import tpu_sc as plsc

P = 32
M = 512
N = 512
IMG = M * N
NP = N + 2
PROWS = 544
IMGP = PROWS * NP
ROWS_PER_CHUNK = 8
C = ROWS_PER_CHUNK * N
CU = 2 * C
NUM_CHUNKS = IMG // C
VECS = C // 16
BLKW = ROWS_PER_CHUNK * NP
NSLOT = 8
WINW = NSLOT * BLKW


def _warp_body(xp_hbm, u_hbm, out_hbm,
               win, ub, outv,
               f00, f01, f10, f11,
               g00b, g01b, g10b, g11b,
               sem_u, sem_pref, sem_g):
    cid = lax.axis_index("c")
    sid = lax.axis_index("s")
    wid = sid * 2 + cid

    pbase = (wid // 8) * IMGP
    ubase = wid * IMG * 2
    obase = wid * IMG

    lanef = lax.broadcasted_iota(jnp.int32, (16,), 0).astype(jnp.float32)
    iota2 = lax.broadcasted_iota(jnp.int32, (16,), 0) * 2
    zerov = jnp.zeros((16,), jnp.float32)

    def coords(t, k):
        offu = t * 32
        idxu = iota2 + offu
        dxl = plsc.load_gather(ub, [idxu])
        dyl = plsc.load_gather(ub, [idxu + 1])
        jxf = ((t & 31) * 16 + 1).astype(jnp.float32)
        iyf = (k * 8 + (t >> 5) + 1).astype(jnp.float32)
        xs = dxl + lanef + jxf
        ys = dyl + iyf
        xs = jnp.minimum(jnp.maximum(xs, 0.0), float(NP - 1))
        ys = jnp.minimum(jnp.maximum(ys, 0.0), float(NP - 1))
        x0 = xs.astype(jnp.int32)
        y0 = ys.astype(jnp.int32)
        wx = xs - x0.astype(jnp.float32)
        wy = ys - y0.astype(jnp.float32)
        return x0, y0, wx, wy

    def zero_body(t, carry):
        win[pl.ds(t * 16, 16)] = zerov
        return carry

    lax.fori_loop(0, WINW // 16 + 1, zero_body, None)
    pltpu.sync_copy(xp_hbm.at[pl.ds(pbase, 3 * BLKW)],
                    win.at[pl.ds(0, 3 * BLKW)])
    pltpu.async_copy(xp_hbm.at[pl.ds(pbase + 3 * BLKW, BLKW)],
                     win.at[pl.ds(3 * BLKW, BLKW)], sem_pref)

    def chunk_body(k, carry):
        pltpu.async_copy(u_hbm.at[pl.ds(ubase + k * CU, CU)], ub,
                         sem_u).wait()

        pltpu.make_async_copy(
            xp_hbm.at[pl.ds(pbase, BLKW)], win.at[pl.ds(0, BLKW)],
            sem_pref).wait()
        blk = k + 4
        slot = blk & (NSLOT - 1)
        pltpu.async_copy(
            xp_hbm.at[pl.ds(pbase + blk * BLKW, BLKW)],
            win.at[pl.ds(slot * BLKW, BLKW)], sem_pref)

        wlo = k * 8 - 24
        whi = k * 8 + 32

        def vec_body(t, missv):
            off = t * 16
            x0, y0, wx, wy = coords(t, k)
            y1 = y0 + 1
            x1 = x0 + 1

            in0 = (y0 >= wlo) & (y0 < whi)
            in1 = (y1 >= wlo) & (y1 < whi)
            miss = ~(in0 & in1)

            lb0 = (y0 & (NSLOT * 8 - 1)) * NP
            lb1 = (y1 & (NSLOT * 8 - 1)) * NP
            v00 = plsc.load_gather(win, [lb0 + x0])
            v01 = plsc.load_gather(win, [lb0 + x1])
            v10 = plsc.load_gather(win, [lb1 + x0])
            v11 = plsc.load_gather(win, [lb1 + x1])

            ox = 1.0 - wx
            oy = 1.0 - wy
            acc = oy * (v00 * ox + v01 * wx) + wy * (v10 * ox + v11 * wx)
            outv[pl.ds(off, 16)] = acc
            return missv | miss.astype(jnp.int32)

        missv = lax.fori_loop(0, VECS, vec_body,
                              jnp.zeros((16,), jnp.int32))
        nmiss = jnp.max(missv)

        @pl.when(nmiss > 0)
        def _fallback():
            def idx_body(t, carry):
                off = t * 16
                x0, y0, _wx, _wy = coords(t, k)
                yb0 = y0 * NP + pbase
                yb1 = yb0 + NP
                f00[pl.ds(off, 16)] = yb0 + x0
                f01[pl.ds(off, 16)] = yb0 + x0 + 1
                f10[pl.ds(off, 16)] = yb1 + x0
                f11[pl.ds(off, 16)] = yb1 + x0 + 1
                return carry

            lax.fori_loop(0, VECS, idx_body, None)

            c0 = pltpu.async_copy(xp_hbm.at[f00], g00b, sem_g)
            c1 = pltpu.async_copy(xp_hbm.at[f01], g01b, sem_g)
            c2 = pltpu.async_copy(xp_hbm.at[f10], g10b, sem_g)
            c3 = pltpu.async_copy(xp_hbm.at[f11], g11b, sem_g)
            c0.wait()
            c1.wait()
            c2.wait()
            c3.wait()

            def mix_body(t, carry):
                off = t * 16
                _x0, _y0, wx, wy = coords(t, k)
                ox = 1.0 - wx
                oy = 1.0 - wy
                s = pl.ds(off, 16)
                acc = (oy * (g00b[s] * ox + g01b[s] * wx)
                       + wy * (g10b[s] * ox + g11b[s] * wx))
                outv[s] = acc
                return carry

            lax.fori_loop(0, VECS, mix_body, None)

        pltpu.sync_copy(outv, out_hbm.at[pl.ds(obase + k * C, C)])
        return carry

    lax.fori_loop(0, NUM_CHUNKS, chunk_body, None)

    pltpu.make_async_copy(xp_hbm.at[pl.ds(pbase, BLKW)],
                          win.at[pl.ds(0, BLKW)], sem_pref).wait()


@jax.jit
def _warp_call(xp_flat, u_flat):
    mesh = plsc.VectorSubcoreMesh(core_axis_name="c", subcore_axis_name="s")
    f = pl.kernel(
        _warp_body,
        out_type=jax.ShapeDtypeStruct((P * IMG,), jnp.float32),
        mesh=mesh,
        compiler_params=pltpu.CompilerParams(needs_layout_passes=False),
        scratch_types=[
            pltpu.VMEM((WINW + 16,), jnp.float32),
            pltpu.VMEM((CU,), jnp.float32),
            pltpu.VMEM((C,), jnp.float32),
            pltpu.VMEM((C,), jnp.int32),
            pltpu.VMEM((C,), jnp.int32),
            pltpu.VMEM((C,), jnp.int32),
            pltpu.VMEM((C,), jnp.int32),
            pltpu.VMEM((C,), jnp.float32),
            pltpu.VMEM((C,), jnp.float32),
            pltpu.VMEM((C,), jnp.float32),
            pltpu.VMEM((C,), jnp.float32),
            pltpu.SemaphoreType.DMA,
            pltpu.SemaphoreType.DMA,
            pltpu.SemaphoreType.DMA,
        ],
    )
    return f(xp_flat, u_flat)


def kernel(x, u):
    xp = jnp.zeros((4, PROWS, NP), jnp.float32)
    xp = xp.at[:, 1:M + 1, 1:N + 1].set(x)
    out = _warp_call(xp.reshape(-1), u.reshape(-1))
    return out.reshape(u.shape[:-1])


# speedup vs baseline: 1.0004x; 1.0004x over previous
import jax
import jax.numpy as jnp
from jax import lax
from jax.experimental import pallas as pl
from jax.experimental.pallas import tpu as pltpu
from jax.experimental.pallas import tpu_sc as plsc

P = 32
M = 512
N = 512
IMG = M * N
NP = 640
PROWS = 544
IMGP = PROWS * NP
ROWS_PER_CHUNK = 8
C = ROWS_PER_CHUNK * N
CU = 2 * C
NUM_CHUNKS = IMG // C
VECS = C // 16
BLKW = ROWS_PER_CHUNK * NP
NSLOT = 8
WINW = NSLOT * BLKW


def _warp_body(xp_hbm, u_hbm, out_hbm,
               win, ub, outv,
               f00, f01, f10, f11,
               g00b, g01b, g10b, g11b,
               sem_u, sem_pref, sem_g):
    cid = lax.axis_index("c")
    sid = lax.axis_index("s")
    wid = sid * 2 + cid

    pbase = (wid // 8) * IMGP
    ubase = wid * IMG * 2
    obase = wid * IMG

    lanef = lax.broadcasted_iota(jnp.int32, (16,), 0).astype(jnp.float32)
    iota2 = lax.broadcasted_iota(jnp.int32, (16,), 0) * 2
    zerov = jnp.zeros((16,), jnp.float32)

    def coords(t, k):
        offu = t * 32
        idxu = iota2 + offu
        dxl = plsc.load_gather(ub, [idxu])
        dyl = plsc.load_gather(ub, [idxu + 1])
        jxf = ((t & 31) * 16 + 1).astype(jnp.float32)
        iyf = (k * 8 + (t >> 5) + 1).astype(jnp.float32)
        xs = dxl + lanef + jxf
        ys = dyl + iyf
        xs = jnp.minimum(jnp.maximum(xs, 0.0), 513.0)
        ys = jnp.minimum(jnp.maximum(ys, 0.0), 513.0)
        x0 = xs.astype(jnp.int32)
        y0 = ys.astype(jnp.int32)
        wx = xs - x0.astype(jnp.float32)
        wy = ys - y0.astype(jnp.float32)
        return x0, y0, wx, wy

    def zero_body(t, carry):
        win[pl.ds(t * 16, 16)] = zerov
        return carry

    lax.fori_loop(0, WINW // 16 + 1, zero_body, None)
    pltpu.sync_copy(xp_hbm.at[pl.ds(pbase, 3 * BLKW)],
                    win.at[pl.ds(0, 3 * BLKW)])
    pltpu.async_copy(xp_hbm.at[pl.ds(pbase + 3 * BLKW, BLKW)],
                     win.at[pl.ds(3 * BLKW, BLKW)], sem_pref)

    def chunk_body(k, carry):
        pltpu.async_copy(u_hbm.at[pl.ds(ubase + k * CU, CU)], ub,
                         sem_u).wait()

        pltpu.make_async_copy(
            xp_hbm.at[pl.ds(pbase, BLKW)], win.at[pl.ds(0, BLKW)],
            sem_pref).wait()
        blk = k + 4
        slot = blk & (NSLOT - 1)
        pltpu.async_copy(
            xp_hbm.at[pl.ds(pbase + blk * BLKW, BLKW)],
            win.at[pl.ds(slot * BLKW, BLKW)], sem_pref)

        wlo = k * 8 - 24
        whi = k * 8 + 32

        def vec_body(t, missv):
            off = t * 16
            x0, y0, wx, wy = coords(t, k)
            y1 = y0 + 1
            x1 = x0 + 1

            in0 = (y0 >= wlo) & (y0 < whi)
            in1 = (y1 >= wlo) & (y1 < whi)
            miss = ~(in0 & in1)

            lb0 = (y0 & (NSLOT * 8 - 1)) * NP
            lb1 = (y1 & (NSLOT * 8 - 1)) * NP
            v00 = plsc.load_gather(win, [lb0 + x0])
            v01 = plsc.load_gather(win, [lb0 + x1])
            v10 = plsc.load_gather(win, [lb1 + x0])
            v11 = plsc.load_gather(win, [lb1 + x1])

            ox = 1.0 - wx
            oy = 1.0 - wy
            acc = oy * (v00 * ox + v01 * wx) + wy * (v10 * ox + v11 * wx)
            outv[pl.ds(off, 16)] = acc
            return missv | miss.astype(jnp.int32)

        missv = lax.fori_loop(0, VECS, vec_body,
                              jnp.zeros((16,), jnp.int32))
        nmiss = jnp.max(missv)

        @pl.when(nmiss > 0)
        def _fallback():
            def idx_body(t, carry):
                off = t * 16
                x0, y0, _wx, _wy = coords(t, k)
                yb0 = y0 * NP + pbase
                yb1 = yb0 + NP
                f00[pl.ds(off, 16)] = yb0 + x0
                f01[pl.ds(off, 16)] = yb0 + x0 + 1
                f10[pl.ds(off, 16)] = yb1 + x0
                f11[pl.ds(off, 16)] = yb1 + x0 + 1
                return carry

            lax.fori_loop(0, VECS, idx_body, None)

            c0 = pltpu.async_copy(xp_hbm.at[f00], g00b, sem_g)
            c1 = pltpu.async_copy(xp_hbm.at[f01], g01b, sem_g)
            c2 = pltpu.async_copy(xp_hbm.at[f10], g10b, sem_g)
            c3 = pltpu.async_copy(xp_hbm.at[f11], g11b, sem_g)
            c0.wait()
            c1.wait()
            c2.wait()
            c3.wait()

            def mix_body(t, carry):
                off = t * 16
                _x0, _y0, wx, wy = coords(t, k)
                ox = 1.0 - wx
                oy = 1.0 - wy
                s = pl.ds(off, 16)
                acc = (oy * (g00b[s] * ox + g01b[s] * wx)
                       + wy * (g10b[s] * ox + g11b[s] * wx))
                outv[s] = acc
                return carry

            lax.fori_loop(0, VECS, mix_body, None)

        pltpu.sync_copy(outv, out_hbm.at[pl.ds(obase + k * C, C)])
        return carry

    lax.fori_loop(0, NUM_CHUNKS, chunk_body, None)

    pltpu.make_async_copy(xp_hbm.at[pl.ds(pbase, BLKW)],
                          win.at[pl.ds(0, BLKW)], sem_pref).wait()


@jax.jit
def _warp_call(xp_flat, u_flat):
    mesh = plsc.VectorSubcoreMesh(core_axis_name="c", subcore_axis_name="s")
    f = pl.kernel(
        _warp_body,
        out_type=jax.ShapeDtypeStruct((P * IMG,), jnp.float32),
        mesh=mesh,
        compiler_params=pltpu.CompilerParams(needs_layout_passes=False),
        scratch_types=[
            pltpu.VMEM((WINW + 16,), jnp.float32),
            pltpu.VMEM((CU,), jnp.float32),
            pltpu.VMEM((C,), jnp.float32),
            pltpu.VMEM((C,), jnp.int32),
            pltpu.VMEM((C,), jnp.int32),
            pltpu.VMEM((C,), jnp.int32),
            pltpu.VMEM((C,), jnp.int32),
            pltpu.VMEM((C,), jnp.float32),
            pltpu.VMEM((C,), jnp.float32),
            pltpu.VMEM((C,), jnp.float32),
            pltpu.VMEM((C,), jnp.float32),
            pltpu.SemaphoreType.DMA,
            pltpu.SemaphoreType.DMA,
            pltpu.SemaphoreType.DMA,
        ],
    )
    return f(xp_flat, u_flat)


def kernel(x, u):
    xp = jnp.zeros((4, PROWS, NP), jnp.float32)
    xp = xp.at[:, 1:M + 1, 1:N + 1].set(x)
    out = _warp_call(xp.reshape(-1), u.reshape(-1))
    return out.reshape(u.shape[:-1])


# speedup vs baseline: 17.8594x; 17.8518x over previous
import jax
import jax.numpy as jnp
from jax import lax
from jax.experimental import pallas as pl
from jax.experimental.pallas import tpu as pltpu
from jax.experimental.pallas import tpu_sc as plsc

P = 32
M = 512
N = 512
IMG = M * N
NP = 640
PROWS = 544
IMGP = PROWS * NP
ROWS_PER_CHUNK = 8
C = ROWS_PER_CHUNK * N
CU = 2 * C
NUM_CHUNKS = IMG // C
VECS = C // 16
BLKW = ROWS_PER_CHUNK * NP
NSLOT = 8
WINW = NSLOT * BLKW


def _warp_body(xp_hbm, dx_hbm, dy_hbm, out_hbm,
               win, dxv, dyv, outv,
               f00, f01, f10, f11,
               g00b, g01b, g10b, g11b,
               sem_u, sem_pref, sem_g):
    cid = lax.axis_index("c")
    sid = lax.axis_index("s")
    wid = sid * 2 + cid

    pbase = (wid // 8) * IMGP
    ubase = wid * IMG
    obase = wid * IMG

    lanef = lax.broadcasted_iota(jnp.int32, (16,), 0).astype(jnp.float32)
    zerov = jnp.zeros((16,), jnp.float32)

    def coords(t, k):
        off = t * 16
        dxl = dxv[pl.ds(off, 16)]
        dyl = dyv[pl.ds(off, 16)]
        jxf = ((t & 31) * 16 + 1).astype(jnp.float32)
        iyf = (k * 8 + (t >> 5) + 1).astype(jnp.float32)
        xs = dxl + lanef + jxf
        ys = dyl + iyf
        xs = jnp.minimum(jnp.maximum(xs, 0.0), 513.0)
        ys = jnp.minimum(jnp.maximum(ys, 0.0), 513.0)
        x0 = xs.astype(jnp.int32)
        y0 = ys.astype(jnp.int32)
        wx = xs - x0.astype(jnp.float32)
        wy = ys - y0.astype(jnp.float32)
        return x0, y0, wx, wy

    def zero_body(t, carry):
        win[pl.ds(t * 16, 16)] = zerov
        return carry

    lax.fori_loop(0, WINW // 16 + 1, zero_body, None)
    pltpu.sync_copy(xp_hbm.at[pl.ds(pbase, 3 * BLKW)],
                    win.at[pl.ds(0, 3 * BLKW)])
    pltpu.async_copy(xp_hbm.at[pl.ds(pbase + 3 * BLKW, BLKW)],
                     win.at[pl.ds(3 * BLKW, BLKW)], sem_pref)

    def chunk_body(k, carry):
        cx = pltpu.async_copy(dx_hbm.at[pl.ds(ubase + k * C, C)], dxv, sem_u)
        cy = pltpu.async_copy(dy_hbm.at[pl.ds(ubase + k * C, C)], dyv, sem_u)
        cx.wait()
        cy.wait()

        pltpu.make_async_copy(
            xp_hbm.at[pl.ds(pbase, BLKW)], win.at[pl.ds(0, BLKW)],
            sem_pref).wait()
        blk = k + 4
        slot = blk & (NSLOT - 1)
        pltpu.async_copy(
            xp_hbm.at[pl.ds(pbase + blk * BLKW, BLKW)],
            win.at[pl.ds(slot * BLKW, BLKW)], sem_pref)

        wlo = k * 8 - 24
        whi = k * 8 + 32

        def vec_body(t, missv):
            off = t * 16
            x0, y0, wx, wy = coords(t, k)
            y1 = y0 + 1
            x1 = x0 + 1

            in0 = (y0 >= wlo) & (y0 < whi)
            in1 = (y1 >= wlo) & (y1 < whi)
            miss = ~(in0 & in1)

            lb0 = (y0 & (NSLOT * 8 - 1)) * NP
            lb1 = (y1 & (NSLOT * 8 - 1)) * NP
            v00 = plsc.load_gather(win, [lb0 + x0])
            v01 = plsc.load_gather(win, [lb0 + x1])
            v10 = plsc.load_gather(win, [lb1 + x0])
            v11 = plsc.load_gather(win, [lb1 + x1])

            ox = 1.0 - wx
            oy = 1.0 - wy
            acc = oy * (v00 * ox + v01 * wx) + wy * (v10 * ox + v11 * wx)
            outv[pl.ds(off, 16)] = acc
            return missv | miss.astype(jnp.int32)

        missv = lax.fori_loop(0, VECS, vec_body,
                              jnp.zeros((16,), jnp.int32))
        nmiss = jnp.max(missv)

        @pl.when(nmiss > 0)
        def _fallback():
            def idx_body(t, carry):
                off = t * 16
                x0, y0, _wx, _wy = coords(t, k)
                yb0 = y0 * NP + pbase
                yb1 = yb0 + NP
                f00[pl.ds(off, 16)] = yb0 + x0
                f01[pl.ds(off, 16)] = yb0 + x0 + 1
                f10[pl.ds(off, 16)] = yb1 + x0
                f11[pl.ds(off, 16)] = yb1 + x0 + 1
                return carry

            lax.fori_loop(0, VECS, idx_body, None)

            c0 = pltpu.async_copy(xp_hbm.at[f00], g00b, sem_g)
            c1 = pltpu.async_copy(xp_hbm.at[f01], g01b, sem_g)
            c2 = pltpu.async_copy(xp_hbm.at[f10], g10b, sem_g)
            c3 = pltpu.async_copy(xp_hbm.at[f11], g11b, sem_g)
            c0.wait()
            c1.wait()
            c2.wait()
            c3.wait()

            def mix_body(t, carry):
                off = t * 16
                _x0, _y0, wx, wy = coords(t, k)
                ox = 1.0 - wx
                oy = 1.0 - wy
                s = pl.ds(off, 16)
                acc = (oy * (g00b[s] * ox + g01b[s] * wx)
                       + wy * (g10b[s] * ox + g11b[s] * wx))
                outv[s] = acc
                return carry

            lax.fori_loop(0, VECS, mix_body, None)

        pltpu.sync_copy(outv, out_hbm.at[pl.ds(obase + k * C, C)])
        return carry

    lax.fori_loop(0, NUM_CHUNKS, chunk_body, None)

    pltpu.make_async_copy(xp_hbm.at[pl.ds(pbase, BLKW)],
                          win.at[pl.ds(0, BLKW)], sem_pref).wait()


@jax.jit
def _warp_call(xp_flat, dx_flat, dy_flat):
    mesh = plsc.VectorSubcoreMesh(core_axis_name="c", subcore_axis_name="s")
    f = pl.kernel(
        _warp_body,
        out_type=jax.ShapeDtypeStruct((P * IMG,), jnp.float32),
        mesh=mesh,
        compiler_params=pltpu.CompilerParams(needs_layout_passes=False),
        scratch_types=[
            pltpu.VMEM((WINW + 16,), jnp.float32),
            pltpu.VMEM((C,), jnp.float32),
            pltpu.VMEM((C,), jnp.float32),
            pltpu.VMEM((C,), jnp.float32),
            pltpu.VMEM((C,), jnp.int32),
            pltpu.VMEM((C,), jnp.int32),
            pltpu.VMEM((C,), jnp.int32),
            pltpu.VMEM((C,), jnp.int32),
            pltpu.VMEM((C,), jnp.float32),
            pltpu.VMEM((C,), jnp.float32),
            pltpu.VMEM((C,), jnp.float32),
            pltpu.VMEM((C,), jnp.float32),
            pltpu.SemaphoreType.DMA,
            pltpu.SemaphoreType.DMA,
            pltpu.SemaphoreType.DMA,
        ],
    )
    return f(xp_flat, dx_flat, dy_flat)


def kernel(x, u):
    xp = jnp.zeros((4, PROWS, NP), jnp.float32)
    xp = xp.at[:, 1:M + 1, 1:N + 1].set(x)
    out = _warp_call(xp.reshape(-1), u[..., 0].reshape(-1),
                     u[..., 1].reshape(-1))
    return out.reshape(u.shape[:-1])


# speedup vs baseline: 20.3526x; 1.1396x over previous
import jax
import jax.numpy as jnp
from jax import lax
from jax.experimental import pallas as pl
from jax.experimental.pallas import tpu as pltpu
from jax.experimental.pallas import tpu_sc as plsc

P = 32
M = 512
N = 512
IMG = M * N
NP = 640
PROWS = 544
IMGP = PROWS * NP
ROWS_PER_CHUNK = 8
C = ROWS_PER_CHUNK * N
CU = 2 * C
NUM_CHUNKS = IMG // C
VECS = C // 16
BLKW = ROWS_PER_CHUNK * NP
NSLOT = 8
WINW = NSLOT * BLKW


def _warp_body(xp_hbm, dx_hbm, dy_hbm, out_hbm,
               win, dxv0, dyv0, dxv1, dyv1, outv0, outv1,
               f00, f01, f10, f11,
               g00b, g01b, g10b, g11b,
               sem_u0, sem_u1, sem_o0, sem_o1, sem_pref, sem_g):
    cid = lax.axis_index("c")
    sid = lax.axis_index("s")
    wid = sid * 2 + cid

    pbase = (wid // 8) * IMGP
    ubase = wid * IMG
    obase = wid * IMG

    lanef = lax.broadcasted_iota(jnp.int32, (16,), 0).astype(jnp.float32)
    zerov = jnp.zeros((16,), jnp.float32)

    def coords(t, k, dxv, dyv):
        off = t * 16
        dxl = dxv[pl.ds(off, 16)]
        dyl = dyv[pl.ds(off, 16)]
        jxf = ((t & 31) * 16 + 1).astype(jnp.float32)
        iyf = (k * 8 + (t >> 5) + 1).astype(jnp.float32)
        xs = dxl + lanef + jxf
        ys = dyl + iyf
        xs = jnp.minimum(jnp.maximum(xs, 0.0), 513.0)
        ys = jnp.minimum(jnp.maximum(ys, 0.0), 513.0)
        x0 = xs.astype(jnp.int32)
        y0 = ys.astype(jnp.int32)
        wx = xs - x0.astype(jnp.float32)
        wy = ys - y0.astype(jnp.float32)
        return x0, y0, wx, wy

    def zero_body(t, carry):
        win[pl.ds(t * 16, 16)] = zerov
        return carry

    lax.fori_loop(0, WINW // 16 + 1, zero_body, None)
    pltpu.sync_copy(xp_hbm.at[pl.ds(pbase, 3 * BLKW)],
                    win.at[pl.ds(0, 3 * BLKW)])
    pltpu.async_copy(xp_hbm.at[pl.ds(pbase + 3 * BLKW, BLKW)],
                     win.at[pl.ds(3 * BLKW, BLKW)], sem_pref)
    pltpu.async_copy(dx_hbm.at[pl.ds(ubase, C)], dxv0, sem_u0)
    pltpu.async_copy(dy_hbm.at[pl.ds(ubase, C)], dyv0, sem_u0)

    def chunk_k(k, dxv, dyv, dxn, dyn, outv, sem_u, sem_un, sem_o):
        pltpu.make_async_copy(dx_hbm.at[pl.ds(ubase, C)], dxv, sem_u).wait()
        pltpu.make_async_copy(dy_hbm.at[pl.ds(ubase, C)], dyv, sem_u).wait()
        nk = jnp.minimum(k + 1, NUM_CHUNKS - 1)
        pltpu.async_copy(dx_hbm.at[pl.ds(ubase + nk * C, C)], dxn, sem_un)
        pltpu.async_copy(dy_hbm.at[pl.ds(ubase + nk * C, C)], dyn, sem_un)

        pltpu.make_async_copy(
            xp_hbm.at[pl.ds(pbase, BLKW)], win.at[pl.ds(0, BLKW)],
            sem_pref).wait()
        blk = k + 4
        slot = blk & (NSLOT - 1)
        pltpu.async_copy(
            xp_hbm.at[pl.ds(pbase + blk * BLKW, BLKW)],
            win.at[pl.ds(slot * BLKW, BLKW)], sem_pref)

        wlo = k * 8 - 24
        whi = k * 8 + 31

        @pl.when(k >= 2)
        def _():
            pltpu.make_async_copy(outv, out_hbm.at[pl.ds(obase, C)],
                                  sem_o).wait()

        def vec_body(t, missv):
            off = t * 16
            x0, y0, wx, wy = coords(t, k, dxv, dyv)
            y1 = y0 + 1
            x1 = x0 + 1

            miss = ~((y0 >= wlo) & (y0 < whi))

            lb0 = (y0 & (NSLOT * 8 - 1)) * NP
            lb1 = (y1 & (NSLOT * 8 - 1)) * NP
            v00 = plsc.load_gather(win, [lb0 + x0])
            v01 = plsc.load_gather(win, [lb0 + x1])
            v10 = plsc.load_gather(win, [lb1 + x0])
            v11 = plsc.load_gather(win, [lb1 + x1])

            ox = 1.0 - wx
            oy = 1.0 - wy
            acc = oy * (v00 * ox + v01 * wx) + wy * (v10 * ox + v11 * wx)
            outv[pl.ds(off, 16)] = acc
            return missv | miss.astype(jnp.int32)

        missv = lax.fori_loop(0, VECS, vec_body,
                              jnp.zeros((16,), jnp.int32))
        nmiss = jnp.max(missv)

        @pl.when(nmiss > 0)
        def _fallback():
            def idx_body(t, carry):
                off = t * 16
                x0, y0, _wx, _wy = coords(t, k, dxv, dyv)
                yb0 = y0 * NP + pbase
                yb1 = yb0 + NP
                f00[pl.ds(off, 16)] = yb0 + x0
                f01[pl.ds(off, 16)] = yb0 + x0 + 1
                f10[pl.ds(off, 16)] = yb1 + x0
                f11[pl.ds(off, 16)] = yb1 + x0 + 1
                return carry

            lax.fori_loop(0, VECS, idx_body, None)

            c0 = pltpu.async_copy(xp_hbm.at[f00], g00b, sem_g)
            c1 = pltpu.async_copy(xp_hbm.at[f01], g01b, sem_g)
            c2 = pltpu.async_copy(xp_hbm.at[f10], g10b, sem_g)
            c3 = pltpu.async_copy(xp_hbm.at[f11], g11b, sem_g)
            c0.wait()
            c1.wait()
            c2.wait()
            c3.wait()

            def mix_body(t, carry):
                off = t * 16
                _x0, _y0, wx, wy = coords(t, k, dxv, dyv)
                ox = 1.0 - wx
                oy = 1.0 - wy
                s = pl.ds(off, 16)
                acc = (oy * (g00b[s] * ox + g01b[s] * wx)
                       + wy * (g10b[s] * ox + g11b[s] * wx))
                outv[s] = acc
                return carry

            lax.fori_loop(0, VECS, mix_body, None)

        pltpu.async_copy(outv, out_hbm.at[pl.ds(obase + k * C, C)], sem_o)

    def pair_body(m, carry):
        chunk_k(2 * m, dxv0, dyv0, dxv1, dyv1, outv0, sem_u0, sem_u1,
                sem_o0)
        chunk_k(2 * m + 1, dxv1, dyv1, dxv0, dyv0, outv1, sem_u1, sem_u0,
                sem_o1)
        return carry

    lax.fori_loop(0, NUM_CHUNKS // 2, pair_body, None)

    pltpu.make_async_copy(outv0, out_hbm.at[pl.ds(obase, C)], sem_o0).wait()
    pltpu.make_async_copy(outv1, out_hbm.at[pl.ds(obase, C)], sem_o1).wait()
    pltpu.make_async_copy(dx_hbm.at[pl.ds(ubase, C)], dxv0, sem_u0).wait()
    pltpu.make_async_copy(dy_hbm.at[pl.ds(ubase, C)], dyv0, sem_u0).wait()
    pltpu.make_async_copy(xp_hbm.at[pl.ds(pbase, BLKW)],
                          win.at[pl.ds(0, BLKW)], sem_pref).wait()


@jax.jit
def _warp_call(xp_flat, dx_flat, dy_flat):
    mesh = plsc.VectorSubcoreMesh(core_axis_name="c", subcore_axis_name="s")
    f = pl.kernel(
        _warp_body,
        out_type=jax.ShapeDtypeStruct((P * IMG,), jnp.float32),
        mesh=mesh,
        compiler_params=pltpu.CompilerParams(needs_layout_passes=False),
        scratch_types=[
            pltpu.VMEM((WINW + 16,), jnp.float32),
            pltpu.VMEM((C,), jnp.float32),
            pltpu.VMEM((C,), jnp.float32),
            pltpu.VMEM((C,), jnp.float32),
            pltpu.VMEM((C,), jnp.float32),
            pltpu.VMEM((C,), jnp.float32),
            pltpu.VMEM((C,), jnp.float32),
            pltpu.VMEM((C,), jnp.int32),
            pltpu.VMEM((C,), jnp.int32),
            pltpu.VMEM((C,), jnp.int32),
            pltpu.VMEM((C,), jnp.int32),
            pltpu.VMEM((C,), jnp.float32),
            pltpu.VMEM((C,), jnp.float32),
            pltpu.VMEM((C,), jnp.float32),
            pltpu.VMEM((C,), jnp.float32),
            pltpu.SemaphoreType.DMA,
            pltpu.SemaphoreType.DMA,
            pltpu.SemaphoreType.DMA,
            pltpu.SemaphoreType.DMA,
            pltpu.SemaphoreType.DMA,
            pltpu.SemaphoreType.DMA,
        ],
    )
    return f(xp_flat, dx_flat, dy_flat)


def kernel(x, u):
    xp = jnp.zeros((4, PROWS, NP), jnp.float32)
    xp = xp.at[:, 1:M + 1, 1:N + 1].set(x)
    out = _warp_call(xp.reshape(-1), u[..., 0].reshape(-1),
                     u[..., 1].reshape(-1))
    return out.reshape(u.shape[:-1])


# speedup vs baseline: 38.2091x; 1.8774x over previous
import jax
import jax.numpy as jnp
from jax import lax
from jax.experimental import pallas as pl
from jax.experimental.pallas import tpu as pltpu
from jax.experimental.pallas import tpu_sc as plsc

P = 32
M = 512
N = 512
IMG = M * N
NP = 640
PROWS = 544
IMGP = PROWS * NP
ROWS_PER_CHUNK = 8
C = ROWS_PER_CHUNK * N
CU = 2 * C
NUM_CHUNKS = IMG // C
VECS = C // 16
BLKW = ROWS_PER_CHUNK * NP
NSLOT = 8
WINW = NSLOT * BLKW


def _warp_body(xp_hbm, dx_hbm, dy_hbm, out_hbm,
               win, dxv0, dyv0, dxv1, dyv1, outv0, outv1,
               f00, f01, f10, f11,
               g00b, g01b, g10b, g11b,
               sem_u0, sem_u1, sem_o0, sem_o1, sem_pref, sem_g):
    cid = lax.axis_index("c")
    sid = lax.axis_index("s")
    wid = sid * 2 + cid

    pbase = (wid // 8) * IMGP
    ubase = wid * IMG
    obase = wid * IMG

    lanef = lax.broadcasted_iota(jnp.int32, (16,), 0).astype(jnp.float32)
    zerov = jnp.zeros((16,), jnp.float32)

    def coords(t, k, dxv, dyv):
        off = t * 16
        dxl = dxv[pl.ds(off, 16)]
        dyl = dyv[pl.ds(off, 16)]
        jxf = ((t & 31) * 16 + 1).astype(jnp.float32)
        iyf = (k * 8 + (t >> 5) + 1).astype(jnp.float32)
        xs = dxl + lanef + jxf
        ys = dyl + iyf
        xs = jnp.minimum(jnp.maximum(xs, 0.0), 513.0)
        ys = jnp.minimum(jnp.maximum(ys, 0.0), 513.0)
        x0 = xs.astype(jnp.int32)
        y0 = ys.astype(jnp.int32)
        wx = xs - x0.astype(jnp.float32)
        wy = ys - y0.astype(jnp.float32)
        return x0, y0, wx, wy

    def zero_body(t, carry):
        win[pl.ds(t * 16, 16)] = zerov
        return carry

    lax.fori_loop(0, WINW // 16 + 1, zero_body, None)
    pltpu.sync_copy(xp_hbm.at[pl.ds(pbase, 3 * BLKW)],
                    win.at[pl.ds(0, 3 * BLKW)])
    pltpu.async_copy(xp_hbm.at[pl.ds(pbase + 3 * BLKW, BLKW)],
                     win.at[pl.ds(3 * BLKW, BLKW)], sem_pref)
    pltpu.async_copy(dx_hbm.at[pl.ds(ubase, C)], dxv0, sem_u0)
    pltpu.async_copy(dy_hbm.at[pl.ds(ubase, C)], dyv0, sem_u0)

    def chunk_k(k, dxv, dyv, dxn, dyn, outv, sem_u, sem_un, sem_o):
        pltpu.make_async_copy(dx_hbm.at[pl.ds(ubase, C)], dxv, sem_u).wait()
        pltpu.make_async_copy(dy_hbm.at[pl.ds(ubase, C)], dyv, sem_u).wait()
        nk = jnp.minimum(k + 1, NUM_CHUNKS - 1)
        pltpu.async_copy(dx_hbm.at[pl.ds(ubase + nk * C, C)], dxn, sem_un)
        pltpu.async_copy(dy_hbm.at[pl.ds(ubase + nk * C, C)], dyn, sem_un)

        pltpu.make_async_copy(
            xp_hbm.at[pl.ds(pbase, BLKW)], win.at[pl.ds(0, BLKW)],
            sem_pref).wait()
        blk = k + 4
        slot = blk & (NSLOT - 1)
        pltpu.async_copy(
            xp_hbm.at[pl.ds(pbase + blk * BLKW, BLKW)],
            win.at[pl.ds(slot * BLKW, BLKW)], sem_pref)

        wlo = k * 8 - 24
        whi = k * 8 + 31

        @pl.when(k >= 2)
        def _():
            pltpu.make_async_copy(outv, out_hbm.at[pl.ds(obase, C)],
                                  sem_o).wait()

        def vec_body(t, missv):
            off = t * 16
            x0, y0, wx, wy = coords(t, k, dxv, dyv)
            y1 = y0 + 1
            x1 = x0 + 1

            miss = ~((y0 >= wlo) & (y0 < whi))

            lb0 = (y0 & (NSLOT * 8 - 1)) * NP
            lb1 = (y1 & (NSLOT * 8 - 1)) * NP
            v00 = plsc.load_gather(win, [lb0 + x0])
            v01 = plsc.load_gather(win, [lb0 + x1])
            v10 = plsc.load_gather(win, [lb1 + x0])
            v11 = plsc.load_gather(win, [lb1 + x1])

            ox = 1.0 - wx
            oy = 1.0 - wy
            acc = oy * (v00 * ox + v01 * wx) + wy * (v10 * ox + v11 * wx)
            outv[pl.ds(off, 16)] = acc
            return missv | miss.astype(jnp.int32)

        missv = plsc.parallel_loop(
            0, VECS, unroll=4,
            carry=jnp.zeros((16,), jnp.int32))(vec_body)
        nmiss = jnp.max(missv)

        @pl.when(nmiss > 0)
        def _fallback():
            def idx_body(t, carry):
                off = t * 16
                x0, y0, _wx, _wy = coords(t, k, dxv, dyv)
                yb0 = y0 * NP + pbase
                yb1 = yb0 + NP
                f00[pl.ds(off, 16)] = yb0 + x0
                f01[pl.ds(off, 16)] = yb0 + x0 + 1
                f10[pl.ds(off, 16)] = yb1 + x0
                f11[pl.ds(off, 16)] = yb1 + x0 + 1
                return carry

            lax.fori_loop(0, VECS, idx_body, None)

            c0 = pltpu.async_copy(xp_hbm.at[f00], g00b, sem_g)
            c1 = pltpu.async_copy(xp_hbm.at[f01], g01b, sem_g)
            c2 = pltpu.async_copy(xp_hbm.at[f10], g10b, sem_g)
            c3 = pltpu.async_copy(xp_hbm.at[f11], g11b, sem_g)
            c0.wait()
            c1.wait()
            c2.wait()
            c3.wait()

            def mix_body(t, carry):
                off = t * 16
                _x0, _y0, wx, wy = coords(t, k, dxv, dyv)
                ox = 1.0 - wx
                oy = 1.0 - wy
                s = pl.ds(off, 16)
                acc = (oy * (g00b[s] * ox + g01b[s] * wx)
                       + wy * (g10b[s] * ox + g11b[s] * wx))
                outv[s] = acc
                return carry

            lax.fori_loop(0, VECS, mix_body, None)

        pltpu.async_copy(outv, out_hbm.at[pl.ds(obase + k * C, C)], sem_o)

    def pair_body(m, carry):
        chunk_k(2 * m, dxv0, dyv0, dxv1, dyv1, outv0, sem_u0, sem_u1,
                sem_o0)
        chunk_k(2 * m + 1, dxv1, dyv1, dxv0, dyv0, outv1, sem_u1, sem_u0,
                sem_o1)
        return carry

    lax.fori_loop(0, NUM_CHUNKS // 2, pair_body, None)

    pltpu.make_async_copy(outv0, out_hbm.at[pl.ds(obase, C)], sem_o0).wait()
    pltpu.make_async_copy(outv1, out_hbm.at[pl.ds(obase, C)], sem_o1).wait()
    pltpu.make_async_copy(dx_hbm.at[pl.ds(ubase, C)], dxv0, sem_u0).wait()
    pltpu.make_async_copy(dy_hbm.at[pl.ds(ubase, C)], dyv0, sem_u0).wait()
    pltpu.make_async_copy(xp_hbm.at[pl.ds(pbase, BLKW)],
                          win.at[pl.ds(0, BLKW)], sem_pref).wait()


@jax.jit
def _warp_call(xp_flat, dx_flat, dy_flat):
    mesh = plsc.VectorSubcoreMesh(core_axis_name="c", subcore_axis_name="s")
    f = pl.kernel(
        _warp_body,
        out_type=jax.ShapeDtypeStruct((P * IMG,), jnp.float32),
        mesh=mesh,
        compiler_params=pltpu.CompilerParams(needs_layout_passes=False),
        scratch_types=[
            pltpu.VMEM((WINW + 16,), jnp.float32),
            pltpu.VMEM((C,), jnp.float32),
            pltpu.VMEM((C,), jnp.float32),
            pltpu.VMEM((C,), jnp.float32),
            pltpu.VMEM((C,), jnp.float32),
            pltpu.VMEM((C,), jnp.float32),
            pltpu.VMEM((C,), jnp.float32),
            pltpu.VMEM((C,), jnp.int32),
            pltpu.VMEM((C,), jnp.int32),
            pltpu.VMEM((C,), jnp.int32),
            pltpu.VMEM((C,), jnp.int32),
            pltpu.VMEM((C,), jnp.float32),
            pltpu.VMEM((C,), jnp.float32),
            pltpu.VMEM((C,), jnp.float32),
            pltpu.VMEM((C,), jnp.float32),
            pltpu.SemaphoreType.DMA,
            pltpu.SemaphoreType.DMA,
            pltpu.SemaphoreType.DMA,
            pltpu.SemaphoreType.DMA,
            pltpu.SemaphoreType.DMA,
            pltpu.SemaphoreType.DMA,
        ],
    )
    return f(xp_flat, dx_flat, dy_flat)


def kernel(x, u):
    xp = jnp.zeros((4, PROWS, NP), jnp.float32)
    xp = xp.at[:, 1:M + 1, 1:N + 1].set(x)
    out = _warp_call(xp.reshape(-1), u[..., 0].reshape(-1),
                     u[..., 1].reshape(-1))
    return out.reshape(u.shape[:-1])
